# Initial kernel scaffold; baseline (speedup 1.0000x reference)
import jax
import jax.numpy as jnp
from jax.experimental import pallas as pl


def kernel(x, edge_index, edge_type, basis, comp, w_self):
    raise NotImplementedError("write your pallas kernel here")


# speedup vs baseline: 7.6262x; 7.6262x over previous
import functools

import jax
import jax.numpy as jnp
from jax import lax
from jax.experimental import pallas as pl
from jax.experimental.pallas import tpu as pltpu, tpu_sc as plsc

N = 10000
E = 320000
DIN = 128
DOUT = 128
NB = 4
NR = 16

NCORES = 2
NSUB = 16
HF = DIN // 2
NBLK = 25
BN = N // NBLK
GROW = BN * NR
SLAB_ROWS = 6528
ZSH = SLAB_ROWS // NSUB
EP = E // NSUB
CHUNK = 128
CAP = EP + CHUNK + 32
CW = 16
XSH = N // NSUB


def _sc_aggregate(x_lo, x_hi, src, dst, etype):
    mesh = plsc.VectorSubcoreMesh(core_axis_name="c", subcore_axis_name="s",
                                  num_cores=NCORES, num_subcores=NSUB)

    @functools.partial(
        pl.kernel,
        out_type=(
            jax.ShapeDtypeStruct((NR * N, HF), jnp.float32),
            jax.ShapeDtypeStruct((NR * N, HF), jnp.float32),
            jax.ShapeDtypeStruct((NR * N, CW), jnp.float32),
        ),
        mesh=mesh,
        compiler_params=pltpu.CompilerParams(needs_layout_passes=False,
                                             use_tc_tiling_on_sc=False),
        scratch_types=[
            pltpu.VMEM((EP,), jnp.int32),
            pltpu.VMEM((CAP,), jnp.int32),
            pltpu.VMEM((CHUNK,), jnp.int32),
            pltpu.VMEM((CHUNK,), jnp.int32),
            pltpu.VMEM((CHUNK, HF), jnp.float32),
            pltpu.VMEM((32, HF), jnp.float32),
            pltpu.VMEM((64, CW), jnp.float32),
            pltpu.VMEM((CHUNK, CW), jnp.float32),
            pltpu.VMEM_SHARED((N, HF), jnp.float32),
            pltpu.VMEM_SHARED((SLAB_ROWS, HF), jnp.float32),
            pltpu.VMEM_SHARED((SLAB_ROWS, CW), jnp.float32),
            pltpu.SemaphoreType.DMA,
        ],
    )
    def agg(xlo_hbm, xhi_hbm, src_hbm, dst_hbm, et_hbm,
            slo_out, shi_out, cnt_out,
            b_pk, b_pkc, loc_chunk, src_chunk,
            rows_v, zbuf, zcnt, ones_v, x_sp, slab, cnt_slab, sem):
        c = lax.axis_index("c")
        s = lax.axis_index("s")
        e0 = s * EP

        zvec = jnp.zeros((16,), jnp.float32)
        onev = jnp.ones((16,), jnp.float32)

        def zrow(i, _):
            for m in range(HF // 16):
                zbuf[i, pl.ds(m * 16, 16)] = zvec
            return 0
        lax.fori_loop(0, 32, zrow, 0)

        def zrow2(i, _):
            zcnt[i, :] = zvec
            return 0
        lax.fori_loop(0, 64, zrow2, 0)

        def orow(i, _):
            ones_v[i, :] = onev
            return 0
        lax.fori_loop(0, CHUNK, orow, 0)

        @pl.when(c == 0)
        def _():
            pltpu.sync_copy(xlo_hbm.at[pl.ds(s * XSH, XSH)],
                            x_sp.at[pl.ds(s * XSH, XSH)])

        @pl.when(c == 1)
        def _():
            pltpu.sync_copy(xhi_hbm.at[pl.ds(s * XSH, XSH)],
                            x_sp.at[pl.ds(s * XSH, XSH)])

        pltpu.sync_copy(dst_hbm.at[pl.ds(e0, EP)], b_pk)
        pltpu.sync_copy(et_hbm.at[pl.ds(e0, EP)], b_pkc.at[pl.ds(0, EP)])

        def pack1(j, _):
            o = j * 16
            b_pk[pl.ds(o, 16)] = (b_pk[pl.ds(o, 16)]
                                  | (b_pkc[pl.ds(o, 16)] << 14))
            return 0
        lax.fori_loop(0, EP // 16, pack1, 0)
        pltpu.sync_copy(src_hbm.at[pl.ds(e0, EP)], b_pkc.at[pl.ds(0, EP)])

        def pack2(j, _):
            o = j * 16
            b_pk[pl.ds(o, 16)] = (b_pk[pl.ds(o, 16)]
                                  | (b_pkc[pl.ds(o, 16)] << 18))
            return 0
        lax.fori_loop(0, EP // 16, pack2, 0)
        plsc.subcore_barrier()

        def one_pass(k, _):
            d_lo = k * BN
            count_here = (k % NCORES) == c

            r0 = s * ZSH
            for m in range(ZSH // 32):
                pltpu.sync_copy(zbuf, slab.at[pl.ds(r0 + m * 32, 32)])
            pltpu.sync_copy(zbuf.at[pl.ds(0, ZSH % 32)],
                            slab.at[pl.ds(r0 + (ZSH // 32) * 32, ZSH % 32)])

            @pl.when(count_here)
            def _():
                for m in range(ZSH // 64):
                    pltpu.sync_copy(zcnt, cnt_slab.at[pl.ds(r0 + m * 64, 64)])
                pltpu.sync_copy(zcnt.at[pl.ds(0, ZSH % 64)],
                                cnt_slab.at[pl.ds(r0 + (ZSH // 64) * 64,
                                                  ZSH % 64)])
            plsc.subcore_barrier()

            def compact(j, ptr):
                o = j * 16
                v = b_pk[pl.ds(o, 16)]
                d16 = v & 0x3FFF
                m = (d16 >= d_lo) & (d16 < d_lo + BN)
                mi = m.astype(jnp.int32)
                pos = plsc.cumsum(mi) + (ptr - 1)
                plsc.store_scatter(b_pkc, [pos], v, mask=m)
                return ptr + jnp.sum(mi)
            total = lax.fori_loop(0, EP // 16, compact, 0)

            pad_pk = jnp.full((16,), 0, jnp.int32) + (d_lo + GROW)
            for m in range(CHUNK // 16):
                b_pkc[pl.ds(total + m * 16, 16)] = pad_pk

            nchunks = (total + CHUNK - 1) // CHUNK

            def fire(i, _):
                o = i * CHUNK
                for m in range(CHUNK // 16):
                    v = b_pkc[pl.ds(o + m * 16, 16)]
                    et = (v >> 14) & 0xF
                    loc_chunk[pl.ds(m * 16, 16)] = ((v & 0x3FFF) - d_lo
                                                    + et * BN)
                    src_chunk[pl.ds(m * 16, 16)] = (
                        lax.shift_right_logical(v, 18))
                pltpu.async_copy(x_sp.at[src_chunk], rows_v, sem).wait()
                pltpu.sync_copy(rows_v, slab.at[loc_chunk], add=True)

                @pl.when(count_here)
                def _():
                    pltpu.sync_copy(ones_v, cnt_slab.at[loc_chunk], add=True)
                return 0
            lax.fori_loop(0, nchunks, fire, 0)
            plsc.subcore_barrier()

            q0 = s * BN

            @pl.when(c == 0)
            def _():
                pltpu.sync_copy(slab.at[pl.ds(q0, BN)],
                                slo_out.at[pl.ds(s * N + k * BN, BN)])

            @pl.when(c == 1)
            def _():
                pltpu.sync_copy(slab.at[pl.ds(q0, BN)],
                                shi_out.at[pl.ds(s * N + k * BN, BN)])

            @pl.when(count_here)
            def _():
                pltpu.sync_copy(cnt_slab.at[pl.ds(q0, BN)],
                                cnt_out.at[pl.ds(s * N + k * BN, BN)])
            plsc.subcore_barrier()
            return 0

        lax.fori_loop(0, NBLK, one_pass, 0)

    return agg(x_lo, x_hi, src, dst, etype)


BT = 400


def _tc_body(slo_ref, shi_ref, cnt_ref, x_ref, basis_ref, comp_ref,
             wself_ref, o_ref):
    r = pl.program_id(1)
    cnt = cnt_ref[0][:, 0:1]
    inv = 1.0 / jnp.maximum(cnt, 1.0)
    h_lo = slo_ref[0] * inv
    h_hi = shi_ref[0] * inv
    cw = comp_ref[pl.ds(r, 1), :]
    w = (cw[0, 0] * basis_ref[0] + cw[0, 1] * basis_ref[1]
         + cw[0, 2] * basis_ref[2] + cw[0, 3] * basis_ref[3])
    part = (jnp.dot(h_lo, w[:HF, :], preferred_element_type=jnp.float32)
            + jnp.dot(h_hi, w[HF:, :], preferred_element_type=jnp.float32))

    @pl.when(r == 0)
    def _():
        o_ref[...] = part + jnp.dot(x_ref[...], wself_ref[...],
                                    preferred_element_type=jnp.float32)

    @pl.when(r != 0)
    def _():
        o_ref[...] += part


def _tc_dense(slo3, shi3, cnt3, x, basis, comp, w_self):
    grid = (N // BT, NR)
    return pl.pallas_call(
        _tc_body,
        grid=grid,
        in_specs=[
            pl.BlockSpec((1, BT, HF), lambda i, r: (r, i, 0)),
            pl.BlockSpec((1, BT, HF), lambda i, r: (r, i, 0)),
            pl.BlockSpec((1, BT, CW), lambda i, r: (r, i, 0)),
            pl.BlockSpec((BT, DIN), lambda i, r: (i, 0)),
            pl.BlockSpec((NB, DIN, DOUT), lambda i, r: (0, 0, 0)),
            pl.BlockSpec((NR, NB), lambda i, r: (0, 0)),
            pl.BlockSpec((DIN, DOUT), lambda i, r: (0, 0)),
        ],
        out_specs=pl.BlockSpec((BT, DOUT), lambda i, r: (i, 0)),
        out_shape=jax.ShapeDtypeStruct((N, DOUT), jnp.float32),
    )(slo3, shi3, cnt3, x, basis, comp, w_self)


def kernel(x, edge_index, edge_type, basis, comp, w_self):
    src = edge_index[0]
    dst = edge_index[1]
    x_lo = x[:, :HF]
    x_hi = x[:, HF:]
    s_lo, s_hi, cnt = _sc_aggregate(x_lo, x_hi, src, dst, edge_type)
    slo3 = s_lo.reshape(NR, N, HF)
    shi3 = s_hi.reshape(NR, N, HF)
    cnt3 = cnt.reshape(NR, N, CW)
    return _tc_dense(slo3, shi3, cnt3, x, basis, comp, w_self)
